# baseline (speedup 1.0000x reference)
import functools

import jax
import jax.numpy as jnp
from jax import lax
from jax.experimental import pallas as pl
from jax.experimental.pallas import tpu as pltpu
from jax.experimental.pallas import tpu_sc as plsc

B = 1024
A = 32
F_SIZE = 41024
H1 = 256
H2 = 32
LANES = 16
NGROUP = B // LANES
ZERO_SLOT = F_SIZE
ROW_BUF = F_SIZE + LANES


def _sc_accumulate(W1, idx_t):
    info = plsc.get_sparse_core_info()
    nc, ns = info.num_cores, info.num_subcores
    nw = nc * ns
    rows_per_w = H1 // nw

    mesh = plsc.VectorSubcoreMesh(core_axis_name="c", subcore_axis_name="s")

    @functools.partial(
        pl.kernel,
        out_type=jax.ShapeDtypeStruct((H1, B), jnp.float32),
        mesh=mesh,
        compiler_params=pltpu.CompilerParams(
            needs_layout_passes=False, use_tc_tiling_on_sc=False),
        scratch_types=[
            pltpu.VMEM((NGROUP, A, LANES), jnp.int32),
            pltpu.VMEM((ROW_BUF,), jnp.float32),
            pltpu.VMEM((ROW_BUF,), jnp.float32),
            pltpu.VMEM((B,), jnp.float32),
            pltpu.VMEM((B,), jnp.float32),
            pltpu.SemaphoreType.DMA,
            pltpu.SemaphoreType.DMA,
            pltpu.SemaphoreType.DMA,
        ],
    )
    def sc_kernel(w1_hbm, idx_hbm, out_hbm,
                  idx_v, row0, row1, orow0, orow1, sem_idx, sem_in, sem_out):
        wid = lax.axis_index("s") * nc + lax.axis_index("c")
        h0 = wid * rows_per_w

        cp_idx = pltpu.async_copy(idx_hbm, idx_v, sem_idx)

        zeros16 = jnp.zeros((LANES,), jnp.float32)
        row0[pl.ds(F_SIZE, LANES)] = zeros16
        row1[pl.ds(F_SIZE, LANES)] = zeros16

        rows = [row0, row1]
        orows = [orow0, orow1]

        in_flight = pltpu.async_copy(
            w1_hbm.at[h0], rows[0].at[pl.ds(0, F_SIZE)], sem_in)
        cp_idx.wait()

        out_flight = [None, None]
        for r in range(rows_per_w):
            buf = rows[r % 2]
            orow = orows[r % 2]
            in_flight.wait()
            if r + 1 < rows_per_w:
                in_flight = pltpu.async_copy(
                    w1_hbm.at[h0 + r + 1],
                    rows[(r + 1) % 2].at[pl.ds(0, F_SIZE)], sem_in)
            if out_flight[r % 2] is not None:
                out_flight[r % 2].wait()

            @plsc.parallel_loop(0, NGROUP, unroll=2)
            def g_body(g, buf=buf, orow=orow):
                accs = [jnp.zeros((LANES,), jnp.float32) for _ in range(4)]
                for a in range(A):
                    accs[a % 4] = accs[a % 4] + plsc.load_gather(
                        buf, [idx_v[g, a]])
                orow[pl.ds(g * LANES, LANES)] = (
                    (accs[0] + accs[1]) + (accs[2] + accs[3]))

            out_flight[r % 2] = pltpu.async_copy(
                orow, out_hbm.at[h0 + r], sem_out)

        for of in out_flight:
            if of is not None:
                of.wait()

    return sc_kernel(W1, idx_t)


def _mlp_body(acc_ref, b1_ref, w2_ref, b2_ref, w3_ref, b3_ref, out_ref):
    h1 = jnp.clip(acc_ref[:] + b1_ref[:], 0.0, 1.0)
    h2 = jnp.dot(w2_ref[:], h1, preferred_element_type=jnp.float32)
    h2 = jnp.clip(h2 + b2_ref[:], 0.0, 1.0)
    o = jnp.dot(w3_ref[:], h2, preferred_element_type=jnp.float32) + b3_ref[:]
    out_ref[:] = jnp.tanh(o)


def _mlp(accT, b1, W2, b2, W3, b3):
    return pl.pallas_call(
        _mlp_body,
        out_shape=jax.ShapeDtypeStruct((1, B), jnp.float32),
    )(accT, b1.reshape(H1, 1), W2, b2.reshape(H2, 1), W3, b3.reshape(1, 1))


@jax.jit
def kernel(active_indices, batch_mode, W1, b1, W2, b2, W3, b3):
    idx = active_indices
    eq = idx[:, :, None] == idx[:, None, :]
    earlier = jnp.tril(jnp.ones((A, A), jnp.bool_), k=-1)
    is_dup = jnp.any(eq & earlier[None], axis=-1)
    dead = is_dup | (idx < 0)
    idx_f = jnp.where(dead, ZERO_SLOT, idx).astype(jnp.int32)
    idx_t = idx_f.reshape(NGROUP, LANES, A).transpose(0, 2, 1)
    accT = _sc_accumulate(W1, idx_t)
    out = _mlp(accT, b1, W2, b2, W3, b3)
    return out.reshape(B)


# speedup vs baseline: 1.0180x; 1.0180x over previous
import functools

import jax
import jax.numpy as jnp
from jax import lax
from jax.experimental import pallas as pl
from jax.experimental.pallas import tpu as pltpu
from jax.experimental.pallas import tpu_sc as plsc

B = 1024
A = 32
F_SIZE = 41024
H1 = 256
H2 = 32
LANES = 16
NGROUP = B // LANES
ZERO_SLOT = F_SIZE
ROW_BUF = F_SIZE + LANES


def _sc_accumulate(W1, idx_t):
    info = plsc.get_sparse_core_info()
    nc, ns = info.num_cores, info.num_subcores
    nw = nc * ns
    rows_per_w = H1 // nw

    mesh = plsc.VectorSubcoreMesh(core_axis_name="c", subcore_axis_name="s")

    @functools.partial(
        pl.kernel,
        out_type=jax.ShapeDtypeStruct((H1, B), jnp.float32),
        mesh=mesh,
        compiler_params=pltpu.CompilerParams(
            needs_layout_passes=False, use_tc_tiling_on_sc=False),
        scratch_types=[
            pltpu.VMEM((NGROUP, A, LANES), jnp.int32),
            pltpu.VMEM((ROW_BUF,), jnp.float32),
            pltpu.VMEM((ROW_BUF,), jnp.float32),
            pltpu.VMEM((B,), jnp.float32),
            pltpu.VMEM((B,), jnp.float32),
            pltpu.SemaphoreType.DMA,
            pltpu.SemaphoreType.DMA,
            pltpu.SemaphoreType.DMA,
        ],
    )
    def sc_kernel(w1_hbm, idx_hbm, out_hbm,
                  idx_v, row0, row1, orow0, orow1, sem_idx, sem_in, sem_out):
        wid = lax.axis_index("s") * nc + lax.axis_index("c")
        h0 = wid * rows_per_w

        cp_idx = pltpu.async_copy(idx_hbm, idx_v, sem_idx)

        zeros16 = jnp.zeros((LANES,), jnp.float32)
        row0[pl.ds(F_SIZE, LANES)] = zeros16
        row1[pl.ds(F_SIZE, LANES)] = zeros16

        rows = [row0, row1]
        orows = [orow0, orow1]

        in_flight = pltpu.async_copy(
            w1_hbm.at[pl.ds(h0 * F_SIZE, F_SIZE)],
            rows[0].at[pl.ds(0, F_SIZE)], sem_in)
        cp_idx.wait()

        out_flight = [None, None]
        for r in range(rows_per_w):
            buf = rows[r % 2]
            orow = orows[r % 2]
            in_flight.wait()
            if r + 1 < rows_per_w:
                in_flight = pltpu.async_copy(
                    w1_hbm.at[pl.ds((h0 + r + 1) * F_SIZE, F_SIZE)],
                    rows[(r + 1) % 2].at[pl.ds(0, F_SIZE)], sem_in)
            if out_flight[r % 2] is not None:
                out_flight[r % 2].wait()

            @plsc.parallel_loop(0, NGROUP)
            def g_body(g, buf=buf, orow=orow):
                accs = [jnp.zeros((LANES,), jnp.float32) for _ in range(4)]
                for a in range(A):
                    accs[a % 4] = accs[a % 4] + plsc.load_gather(
                        buf, [idx_v[g, a]])
                orow[pl.ds(g * LANES, LANES)] = (
                    (accs[0] + accs[1]) + (accs[2] + accs[3]))

            out_flight[r % 2] = pltpu.async_copy(
                orow, out_hbm.at[h0 + r], sem_out)

        for of in out_flight:
            if of is not None:
                of.wait()

    return sc_kernel(W1, idx_t)


def _mlp_body(acc_ref, b1_ref, w2_ref, b2_ref, w3_ref, b3_ref, out_ref):
    h1 = jnp.clip(acc_ref[:] + b1_ref[:], 0.0, 1.0)
    h2 = jnp.dot(w2_ref[:], h1, preferred_element_type=jnp.float32)
    h2 = jnp.clip(h2 + b2_ref[:], 0.0, 1.0)
    o = jnp.dot(w3_ref[:], h2, preferred_element_type=jnp.float32) + b3_ref[:]
    out_ref[:] = jnp.tanh(o)


def _mlp(accT, b1, W2, b2, W3, b3):
    return pl.pallas_call(
        _mlp_body,
        out_shape=jax.ShapeDtypeStruct((1, B), jnp.float32),
    )(accT, b1.reshape(H1, 1), W2, b2.reshape(H2, 1), W3, b3.reshape(1, 1))


@jax.jit
def kernel(active_indices, batch_mode, W1, b1, W2, b2, W3, b3):
    idx = active_indices
    eq = idx[:, :, None] == idx[:, None, :]
    earlier = jnp.tril(jnp.ones((A, A), jnp.bool_), k=-1)
    is_dup = jnp.any(eq & earlier[None], axis=-1)
    dead = is_dup | (idx < 0)
    idx_f = jnp.where(dead, ZERO_SLOT, idx).astype(jnp.int32)
    idx_t = idx_f.reshape(NGROUP, LANES, A).transpose(0, 2, 1)
    accT = _sc_accumulate(W1.reshape(H1 * F_SIZE), idx_t)
    out = _mlp(accT, b1, W2, b2, W3, b3)
    return out.reshape(B)


# speedup vs baseline: 1.0607x; 1.0419x over previous
import functools

import jax
import jax.numpy as jnp
from jax import lax
from jax.experimental import pallas as pl
from jax.experimental.pallas import tpu as pltpu
from jax.experimental.pallas import tpu_sc as plsc

B = 1024
A = 32
F_SIZE = 41024
H1 = 256
H2 = 32
LANES = 16
NVEC = H1 // LANES


def _sc_accumulate(W1T, idx_flat, val_flat):
    info = plsc.get_sparse_core_info()
    nc, ns = info.num_cores, info.num_subcores
    nw = nc * ns
    bpw = B // nw

    mesh = plsc.VectorSubcoreMesh(core_axis_name="c", subcore_axis_name="s")

    @functools.partial(
        pl.kernel,
        out_type=jax.ShapeDtypeStruct((B * H1,), jnp.float32),
        mesh=mesh,
        scratch_types=[
            pltpu.VMEM((bpw * A,), jnp.int32),
            pltpu.VMEM((bpw * A,), jnp.float32),
            pltpu.VMEM((A, H1), jnp.float32),
            pltpu.VMEM((A, H1), jnp.float32),
            pltpu.VMEM((bpw * H1,), jnp.float32),
            pltpu.SemaphoreType.DMA,
            pltpu.SemaphoreType.DMA,
            pltpu.SemaphoreType.DMA,
        ],
    )
    def sc_kernel(w1t_hbm, idx_hbm, val_hbm, out_hbm,
                  idx_v, val_v, buf0, buf1, ostage, sem0, sem1, sem_out):
        wid = lax.axis_index("s") * nc + lax.axis_index("c")
        base = wid * (bpw * A)

        pltpu.async_copy(idx_hbm.at[pl.ds(base, bpw * A)], idx_v, sem0).wait()
        pltpu.async_copy(val_hbm.at[pl.ds(base, bpw * A)], val_v, sem1).wait()

        def fire(b, buf, sem):
            return pltpu.async_copy(
                w1t_hbm.at[idx_v.at[pl.ds(b * A, A)]], buf, sem)

        def accumulate(b, buf):
            v0 = val_v[pl.ds(b * A, LANES)]
            v1 = val_v[pl.ds(b * A + LANES, LANES)]
            accs = [jnp.zeros((LANES,), jnp.float32) for _ in range(NVEC)]
            for a in range(A):
                w = (v0 if a < LANES else v1)[a % LANES]
                for v in range(NVEC):
                    accs[v] = accs[v] + w * buf[a, pl.ds(v * LANES, LANES)]
            for v in range(NVEC):
                ostage[pl.ds(b * H1 + v * LANES, LANES)] = accs[v]

        def pair_body(k, _):
            b0 = k * 2
            cp0 = fire(b0, buf0, sem0)
            cp1 = fire(b0 + 1, buf1, sem1)
            cp0.wait()
            accumulate(b0, buf0)
            cp1.wait()
            accumulate(b0 + 1, buf1)
            return 0

        lax.fori_loop(0, bpw // 2, pair_body, 0)

        pltpu.async_copy(
            ostage, out_hbm.at[pl.ds(wid * (bpw * H1), bpw * H1)],
            sem_out).wait()

    return sc_kernel(W1T, idx_flat, val_flat)


def _mlp_body(acc_ref, b1_ref, w2t_ref, b2_ref, w3_ref, b3_ref, out_ref):
    h1 = jnp.clip(acc_ref[:] + b1_ref[:], 0.0, 1.0)
    h2 = jnp.dot(h1, w2t_ref[:], preferred_element_type=jnp.float32)
    h2 = jnp.clip(h2 + b2_ref[:], 0.0, 1.0)
    o = jnp.sum(h2 * w3_ref[:], axis=1, keepdims=True) + b3_ref[0, 0]
    out_ref[:] = jnp.tanh(o)


def _mlp(acc, b1, W2, b2, W3, b3):
    return pl.pallas_call(
        _mlp_body,
        out_shape=jax.ShapeDtypeStruct((B, 1), jnp.float32),
    )(acc, b1.reshape(1, H1), W2.T, b2.reshape(1, H2), W3.reshape(1, H2),
      b3.reshape(1, 1))


@jax.jit
def kernel(active_indices, batch_mode, W1, b1, W2, b2, W3, b3):
    idx = active_indices
    eq = idx[:, :, None] == idx[:, None, :]
    earlier = jnp.tril(jnp.ones((A, A), jnp.bool_), k=-1)
    is_dup = jnp.any(eq & earlier[None], axis=-1)
    dead = is_dup | (idx < 0)
    val = jnp.where(dead, 0.0, 1.0).astype(jnp.float32)
    idx_f = jnp.where(dead, 0, idx).astype(jnp.int32)

    acc_flat = _sc_accumulate(W1.T, idx_f.reshape(B * A), val.reshape(B * A))
    out = _mlp(acc_flat.reshape(B, H1), b1, W2, b2, W3, b3)
    return out.reshape(B)


# speedup vs baseline: 1.1312x; 1.0665x over previous
import functools

import jax
import jax.numpy as jnp
from jax import lax
from jax.experimental import pallas as pl
from jax.experimental.pallas import tpu as pltpu
from jax.experimental.pallas import tpu_sc as plsc

B = 1024
A = 32
F_SIZE = 41024
H1 = 256
H2 = 32
LANES = 16
NVEC = H1 // LANES


def _sc_accumulate(W1T, idx_flat, val_flat):
    info = plsc.get_sparse_core_info()
    nc, ns = info.num_cores, info.num_subcores
    nw = nc * ns
    bpw = B // nw

    mesh = plsc.VectorSubcoreMesh(core_axis_name="c", subcore_axis_name="s")

    @functools.partial(
        pl.kernel,
        out_type=jax.ShapeDtypeStruct((B * H1,), jnp.float32),
        mesh=mesh,
        scratch_types=[
            pltpu.VMEM((bpw * A,), jnp.int32),
            pltpu.VMEM((bpw * A,), jnp.float32),
            pltpu.VMEM((A, H1), jnp.float32),
            pltpu.VMEM((A, H1), jnp.float32),
            pltpu.VMEM((A, H1), jnp.float32),
            pltpu.VMEM((A, H1), jnp.float32),
            pltpu.VMEM((bpw * H1,), jnp.float32),
            pltpu.SemaphoreType.DMA,
            pltpu.SemaphoreType.DMA,
            pltpu.SemaphoreType.DMA,
            pltpu.SemaphoreType.DMA,
            pltpu.SemaphoreType.DMA,
        ],
    )
    def sc_kernel(w1t_hbm, idx_hbm, val_hbm, out_hbm,
                  idx_v, val_v, buf0, buf1, buf2, buf3, ostage,
                  sem0, sem1, sem2, sem3, sem_out):
        wid = lax.axis_index("s") * nc + lax.axis_index("c")
        base = wid * (bpw * A)

        pltpu.async_copy(idx_hbm.at[pl.ds(base, bpw * A)], idx_v, sem0).wait()
        pltpu.async_copy(val_hbm.at[pl.ds(base, bpw * A)], val_v, sem1).wait()

        bufs = [buf0, buf1, buf2, buf3]
        sems = [sem0, sem1, sem2, sem3]
        DEPTH = 4

        def copy(b, buf, sem):
            off = jnp.minimum(b, bpw - 1) * A
            return pltpu.make_async_copy(
                w1t_hbm.at[idx_v.at[pl.ds(off, A)]], buf, sem)

        def accumulate(b, buf):
            v0 = val_v[pl.ds(b * A, LANES)]
            v1 = val_v[pl.ds(b * A + LANES, LANES)]
            accs = [jnp.zeros((LANES,), jnp.float32) for _ in range(NVEC)]
            for a in range(A):
                w = (v0 if a < LANES else v1)[a % LANES]
                for v in range(NVEC):
                    accs[v] = accs[v] + w * buf[a, pl.ds(v * LANES, LANES)]
            for v in range(NVEC):
                ostage[pl.ds(b * H1 + v * LANES, LANES)] = accs[v]

        for j in range(DEPTH):
            copy(j, bufs[j], sems[j]).start()

        def quad_body(k, _):
            b0 = k * DEPTH
            for j in range(DEPTH):
                b = b0 + j
                copy(b, bufs[j], sems[j]).wait()
                accumulate(b, bufs[j])
                copy(b + DEPTH, bufs[j], sems[j]).start()
            return 0

        lax.fori_loop(0, bpw // DEPTH, quad_body, 0)

        for j in range(DEPTH):
            copy(bpw - 1, bufs[j], sems[j]).wait()

        pltpu.async_copy(
            ostage, out_hbm.at[pl.ds(wid * (bpw * H1), bpw * H1)],
            sem_out).wait()

    return sc_kernel(W1T, idx_flat, val_flat)


def _mlp_body(acc_ref, b1_ref, w2t_ref, b2_ref, w3_ref, b3_ref, out_ref):
    h1 = jnp.clip(acc_ref[:] + b1_ref[:], 0.0, 1.0)
    h2 = jnp.dot(h1, w2t_ref[:], preferred_element_type=jnp.float32)
    h2 = jnp.clip(h2 + b2_ref[:], 0.0, 1.0)
    o = jnp.sum(h2 * w3_ref[:], axis=1, keepdims=True) + b3_ref[0, 0]
    out_ref[:] = jnp.tanh(o)


def _mlp(acc, b1, W2, b2, W3, b3):
    return pl.pallas_call(
        _mlp_body,
        out_shape=jax.ShapeDtypeStruct((B, 1), jnp.float32),
    )(acc, b1.reshape(1, H1), W2.T, b2.reshape(1, H2), W3.reshape(1, H2),
      b3.reshape(1, 1))


@jax.jit
def kernel(active_indices, batch_mode, W1, b1, W2, b2, W3, b3):
    idx = active_indices
    eq = idx[:, :, None] == idx[:, None, :]
    earlier = jnp.tril(jnp.ones((A, A), jnp.bool_), k=-1)
    is_dup = jnp.any(eq & earlier[None], axis=-1)
    dead = is_dup | (idx < 0)
    val = jnp.where(dead, 0.0, 1.0).astype(jnp.float32)
    idx_f = jnp.where(dead, 0, idx).astype(jnp.int32)

    acc_flat = _sc_accumulate(W1.T, idx_f.reshape(B * A), val.reshape(B * A))
    out = _mlp(acc_flat.reshape(B, H1), b1, W2, b2, W3, b3)
    return out.reshape(B)


# speedup vs baseline: 1.5321x; 1.3544x over previous
import functools

import jax
import jax.numpy as jnp
from jax import lax
from jax.experimental import pallas as pl
from jax.experimental.pallas import tpu as pltpu
from jax.experimental.pallas import tpu_sc as plsc

B = 1024
A = 32
F_SIZE = 41024
H1 = 256
H2 = 32
LANES = 16
NVEC = H1 // LANES


GB = 4


def _sc_accumulate(W1T, idx_flat, val_splat):
    info = plsc.get_sparse_core_info()
    nc, ns = info.num_cores, info.num_subcores
    nw = nc * ns
    bpw = B // nw
    ngr = bpw // GB

    mesh = plsc.VectorSubcoreMesh(core_axis_name="c", subcore_axis_name="s")

    @functools.partial(
        pl.kernel,
        out_type=jax.ShapeDtypeStruct((B * H1,), jnp.float32),
        mesh=mesh,
        scratch_types=[
            pltpu.VMEM((bpw * A,), jnp.int32),
            pltpu.VMEM((bpw * A * LANES,), jnp.float32),
            pltpu.VMEM((GB * A, H1), jnp.float32),
            pltpu.VMEM((GB * A, H1), jnp.float32),
            pltpu.VMEM((bpw * H1,), jnp.float32),
            pltpu.SemaphoreType.DMA,
            pltpu.SemaphoreType.DMA,
            pltpu.SemaphoreType.DMA,
        ],
    )
    def sc_kernel(w1t_hbm, idx_hbm, val_hbm, out_hbm,
                  idx_v, val_s, buf0, buf1, ostage, sem0, sem1, sem_out):
        wid = lax.axis_index("s") * nc + lax.axis_index("c")

        pltpu.async_copy(
            idx_hbm.at[pl.ds(wid * (bpw * A), bpw * A)], idx_v, sem0).wait()
        pltpu.async_copy(
            val_hbm.at[pl.ds(wid * (bpw * A * LANES), bpw * A * LANES)],
            val_s, sem1).wait()

        def copy(g, buf, sem):
            off = jnp.minimum(g, ngr - 1) * (GB * A)
            return pltpu.make_async_copy(
                w1t_hbm.at[idx_v.at[pl.ds(off, GB * A)]], buf, sem)

        def accumulate(b, row0, buf):
            def a_body(a, accs):
                w = val_s[pl.ds((b * A + a) * LANES, LANES)]
                return tuple(
                    accs[v] + w * buf[row0 + a, pl.ds(v * LANES, LANES)]
                    for v in range(NVEC))

            accs = lax.fori_loop(
                0, A, a_body,
                tuple(jnp.zeros((LANES,), jnp.float32) for _ in range(NVEC)))
            for v in range(NVEC):
                ostage[pl.ds(b * H1 + v * LANES, LANES)] = accs[v]

        copy(0, buf0, sem0).start()
        copy(1, buf1, sem1).start()

        def pair_body(m, _):
            for par, buf, sem in ((0, buf0, sem0), (1, buf1, sem1)):
                g = m * 2 + par
                copy(g, buf, sem).wait()
                for j in range(GB):
                    accumulate(g * GB + j, j * A, buf)
                copy(g + 2, buf, sem).start()
            return 0

        lax.fori_loop(0, ngr // 2, pair_body, 0)

        copy(ngr - 1, buf0, sem0).wait()
        copy(ngr - 1, buf1, sem1).wait()

        pltpu.async_copy(
            ostage, out_hbm.at[pl.ds(wid * (bpw * H1), bpw * H1)],
            sem_out).wait()

    return sc_kernel(W1T, idx_flat, val_splat)


def _mlp_body(acc_ref, b1_ref, w2t_ref, b2_ref, w3_ref, b3_ref, out_ref):
    h1 = jnp.clip(acc_ref[:] + b1_ref[:], 0.0, 1.0)
    h2 = jnp.dot(h1, w2t_ref[:], preferred_element_type=jnp.float32)
    h2 = jnp.clip(h2 + b2_ref[:], 0.0, 1.0)
    o = jnp.sum(h2 * w3_ref[:], axis=1, keepdims=True) + b3_ref[0, 0]
    out_ref[:] = jnp.tanh(o)


def _mlp(acc, b1, W2, b2, W3, b3):
    return pl.pallas_call(
        _mlp_body,
        out_shape=jax.ShapeDtypeStruct((B, 1), jnp.float32),
    )(acc, b1.reshape(1, H1), W2.T, b2.reshape(1, H2), W3.reshape(1, H2),
      b3.reshape(1, 1))


@jax.jit
def kernel(active_indices, batch_mode, W1, b1, W2, b2, W3, b3):
    idx = active_indices
    eq = idx[:, :, None] == idx[:, None, :]
    earlier = jnp.tril(jnp.ones((A, A), jnp.bool_), k=-1)
    is_dup = jnp.any(eq & earlier[None], axis=-1)
    dead = is_dup | (idx < 0)
    val = jnp.where(dead, 0.0, 1.0).astype(jnp.float32)
    idx_f = jnp.where(dead, 0, idx).astype(jnp.int32)
    val_splat = jnp.broadcast_to(
        val.reshape(B * A, 1), (B * A, LANES)).reshape(B * A * LANES)

    acc_flat = _sc_accumulate(W1.T, idx_f.reshape(B * A), val_splat)
    out = _mlp(acc_flat.reshape(B, H1), b1, W2, b2, W3, b3)
    return out.reshape(B)


# speedup vs baseline: 1.9525x; 1.2744x over previous
import functools

import jax
import jax.numpy as jnp
from jax import lax
from jax.experimental import pallas as pl
from jax.experimental.pallas import tpu as pltpu
from jax.experimental.pallas import tpu_sc as plsc

B = 1024
A = 32
F_SIZE = 41024
H1 = 256
H2 = 32
LANES = 16
NVEC = H1 // LANES


GB = 4


def _sc_accumulate(W1T, idx_flat, val_splat):
    info = plsc.get_sparse_core_info()
    nc, ns = info.num_cores, info.num_subcores
    nw = nc * ns
    bpw = B // nw
    ngr = bpw // GB

    mesh = plsc.VectorSubcoreMesh(core_axis_name="c", subcore_axis_name="s")

    @functools.partial(
        pl.kernel,
        out_type=jax.ShapeDtypeStruct((B * H1,), jnp.float32),
        mesh=mesh,
        scratch_types=[
            pltpu.VMEM((bpw * A,), jnp.int32),
            pltpu.VMEM((bpw * A,), jnp.float32),
            pltpu.VMEM((GB * A, H1), jnp.float32),
            pltpu.VMEM((GB * A, H1), jnp.float32),
            pltpu.VMEM((bpw * H1,), jnp.float32),
            pltpu.SemaphoreType.DMA,
            pltpu.SemaphoreType.DMA,
            pltpu.SemaphoreType.DMA,
        ],
    )
    def sc_kernel(w1t_hbm, idx_hbm, val_hbm, out_hbm,
                  idx_v, val_s, buf0, buf1, ostage, sem0, sem1, sem_out):
        wid = lax.axis_index("s") * nc + lax.axis_index("c")

        pltpu.async_copy(
            idx_hbm.at[pl.ds(wid * (bpw * A), bpw * A)], idx_v, sem0).wait()
        pltpu.async_copy(
            val_hbm.at[pl.ds(wid * (bpw * A), bpw * A)], val_s, sem1).wait()

        def copy(g, buf, sem):
            off = jnp.minimum(g, ngr - 1) * (GB * A)
            return pltpu.make_async_copy(
                w1t_hbm.at[idx_v.at[pl.ds(off, GB * A)]], buf, sem)

        def accumulate(b, row0, buf):
            v0 = val_s[pl.ds(b * A, LANES)]
            v1 = val_s[pl.ds(b * A + LANES, LANES)]
            accs = [jnp.zeros((LANES,), jnp.float32) for _ in range(NVEC)]
            for a in range(A):
                w = (v0 if a < LANES else v1)[a % LANES]
                for v in range(NVEC):
                    accs[v] = accs[v] + w * buf[row0 + a, pl.ds(v * LANES, LANES)]
            for v in range(NVEC):
                ostage[pl.ds(b * H1 + v * LANES, LANES)] = accs[v]

        copy(0, buf0, sem0).start()
        copy(1, buf1, sem1).start()

        def pair_body(m, _):
            for par, buf, sem in ((0, buf0, sem0), (1, buf1, sem1)):
                g = m * 2 + par
                copy(g, buf, sem).wait()

                def j_body(j, _, buf=buf, g=g):
                    accumulate(g * GB + j, j * A, buf)
                    return 0

                lax.fori_loop(0, GB, j_body, 0)
                copy(g + 2, buf, sem).start()
            return 0

        lax.fori_loop(0, ngr // 2, pair_body, 0)

        copy(ngr - 1, buf0, sem0).wait()
        copy(ngr - 1, buf1, sem1).wait()

        pltpu.async_copy(
            ostage, out_hbm.at[pl.ds(wid * (bpw * H1), bpw * H1)],
            sem_out).wait()

    return sc_kernel(W1T, idx_flat, val_splat)


def _mlp_body(acc_ref, b1_ref, w2t_ref, b2_ref, w3_ref, b3_ref, out_ref):
    h1 = jnp.clip(acc_ref[:] + b1_ref[:], 0.0, 1.0)
    h2 = jnp.dot(h1, w2t_ref[:], preferred_element_type=jnp.float32)
    h2 = jnp.clip(h2 + b2_ref[:], 0.0, 1.0)
    o = jnp.sum(h2 * w3_ref[:], axis=1, keepdims=True) + b3_ref[0, 0]
    out_ref[:] = jnp.tanh(o)


def _mlp(acc, b1, W2, b2, W3, b3):
    return pl.pallas_call(
        _mlp_body,
        out_shape=jax.ShapeDtypeStruct((B, 1), jnp.float32),
    )(acc, b1.reshape(1, H1), W2.T, b2.reshape(1, H2), W3.reshape(1, H2),
      b3.reshape(1, 1))


@jax.jit
def kernel(active_indices, batch_mode, W1, b1, W2, b2, W3, b3):
    idx = active_indices
    eq = idx[:, :, None] == idx[:, None, :]
    earlier = jnp.tril(jnp.ones((A, A), jnp.bool_), k=-1)
    is_dup = jnp.any(eq & earlier[None], axis=-1)
    dead = is_dup | (idx < 0)
    val = jnp.where(dead, 0.0, 1.0).astype(jnp.float32)
    idx_f = jnp.where(dead, 0, idx).astype(jnp.int32)

    acc_flat = _sc_accumulate(W1.T, idx_f.reshape(B * A), val.reshape(B * A))
    out = _mlp(acc_flat.reshape(B, H1), b1, W2, b2, W3, b3)
    return out.reshape(B)
